# Initial kernel scaffold; baseline (speedup 1.0000x reference)
import jax
import jax.numpy as jnp
from jax.experimental import pallas as pl


def kernel(logits, val_freqs):
    raise NotImplementedError("write your pallas kernel here")


# speedup vs baseline: 1210.4409x; 1210.4409x over previous
import jax
import jax.numpy as jnp
import numpy as np
from jax.experimental import pallas as pl
from jax.experimental.pallas import tpu as pltpu

_NUM_BINS = 15
_NUM_CLASSES = 19
_NBR = 9
_H = 512
_W = 512
_HT = 64


def _shift_left(x):
    z = jnp.zeros((x.shape[0], 1), x.dtype)
    return jnp.concatenate([x[:, 1:], z], axis=1)


def _shift_right(x):
    z = jnp.zeros((x.shape[0], 1), x.dtype)
    return jnp.concatenate([z, x[:, :-1]], axis=1)


def _body(tbl_ref, logits_ref, top_ref, bot_ref, out_ref):
    h = pl.program_id(1)
    n_h = pl.num_programs(1)
    ht = _HT

    m = logits_ref[0, 0]
    yh = jnp.zeros((ht, _W), jnp.int32)
    mt = top_ref[0, 0, 0]
    yt = jnp.zeros((1, _W), jnp.int32)
    mb = bot_ref[0, 0, 0]
    yb = jnp.zeros((1, _W), jnp.int32)
    for c in range(1, _NUM_CLASSES):
        xc = logits_ref[0, c]
        gt = xc > m
        m = jnp.where(gt, xc, m)
        yh = jnp.where(gt, c, yh)
        xt = top_ref[0, c, 0]
        gtt = xt > mt
        mt = jnp.where(gtt, xt, mt)
        yt = jnp.where(gtt, c, yt)
        xb = bot_ref[0, c, 0]
        gtb = xb > mb
        mb = jnp.where(gtb, xb, mb)
        yb = jnp.where(gtb, c, yb)

    s = jnp.zeros((ht, _W), jnp.float32)
    for c in range(_NUM_CLASSES):
        e = jnp.exp(logits_ref[0, c] - m)
        out_ref[0, c] = e
        s = s + e
    inv_s = 1.0 / s

    grow = h * ht + jax.lax.broadcasted_iota(jnp.int32, (ht, _W), 0)
    gcol = jax.lax.broadcasted_iota(jnp.int32, (ht, _W), 1)
    vert = (1.0 + (grow > 0).astype(jnp.float32)
            + (grow < _H - 1).astype(jnp.float32))
    horiz = (1.0 + (gcol > 0).astype(jnp.float32)
             + (gcol < _W - 1).astype(jnp.float32))
    ones_cnt = vert * horiz - 1.0

    top_valid = h > 0
    bot_valid = h < n_h - 1

    csum = jnp.zeros((ht, _W), jnp.float32)
    for c in range(_NUM_CLASSES):
        p = out_ref[0, c] * inv_s
        bin_i = jnp.clip(jnp.floor(p * float(_NUM_BINS)).astype(jnp.int32),
                         0, _NUM_BINS - 1)

        bmid = (yh == c).astype(jnp.float32)
        bt = jnp.where(jnp.logical_and(top_valid, yt == c), 1.0, 0.0)
        bb = jnp.where(jnp.logical_and(bot_valid, yb == c), 1.0, 0.0)
        ext = jnp.concatenate([bt, bmid, bb], axis=0)
        row3 = ext + _shift_left(ext) + _shift_right(ext)
        vsum = row3[0:ht] + row3[1:ht + 1] + row3[2:ht + 2]
        pos = vsum - bmid
        match = jnp.where(yh == c, pos, ones_cnt - pos)
        agg = match.astype(jnp.int32)

        comb = agg * _NUM_BINS + bin_i
        row = tbl_ref[c]
        src_a = jnp.broadcast_to(row[0:1, :], (ht, 128))
        src_b = jnp.broadcast_to(row[1:2, :], (ht, 128))
        pieces = []
        for j in range(_W // 128):
            idx = comb[:, j * 128:(j + 1) * 128]
            ga = jnp.take_along_axis(src_a, jnp.minimum(idx, 127), axis=1,
                                     mode="promise_in_bounds")
            gb = jnp.take_along_axis(src_b, jnp.clip(idx - 128, 0, 127),
                                     axis=1, mode="promise_in_bounds")
            pieces.append(jnp.where(idx < 128, ga, gb))
        val = jnp.concatenate(pieces, axis=1)
        out_ref[0, c] = val
        csum = csum + val

    inv = 1.0 / jnp.where(csum == 0.0, 1.0, csum)
    for c in range(_NUM_CLASSES):
        out_ref[0, c] = out_ref[0, c] * inv


def kernel(logits, val_freqs):
    b, nc, hh, ww = logits.shape
    n_h = hh // _HT

    flat = val_freqs.reshape(_NUM_CLASSES, _NBR * _NUM_BINS).astype(jnp.float32)
    tbl = jnp.zeros((_NUM_CLASSES, 2, 128), jnp.float32)
    tbl = tbl.at[:, 0, :].set(flat[:, :128])
    tbl = tbl.at[:, 1, :flat.shape[1] - 128].set(flat[:, 128:])

    tidx = np.arange(n_h) * _HT - 1
    tidx[0] = 0
    bidx = np.arange(n_h) * _HT + _HT
    bidx[-1] = hh - 1
    top = logits[:, :, tidx, :].reshape(b, nc, n_h, 1, ww)
    bot = logits[:, :, bidx, :].reshape(b, nc, n_h, 1, ww)

    return pl.pallas_call(
        _body,
        grid=(b, n_h),
        in_specs=[
            pl.BlockSpec((_NUM_CLASSES, 2, 128), lambda b_, h_: (0, 0, 0)),
            pl.BlockSpec((1, _NUM_CLASSES, _HT, _W),
                         lambda b_, h_: (b_, 0, h_, 0)),
            pl.BlockSpec((1, _NUM_CLASSES, 1, 1, _W),
                         lambda b_, h_: (b_, 0, h_, 0, 0)),
            pl.BlockSpec((1, _NUM_CLASSES, 1, 1, _W),
                         lambda b_, h_: (b_, 0, h_, 0, 0)),
        ],
        out_specs=pl.BlockSpec((1, _NUM_CLASSES, _HT, _W),
                               lambda b_, h_: (b_, 0, h_, 0)),
        out_shape=jax.ShapeDtypeStruct((b, nc, hh, ww), jnp.float32),
        compiler_params=pltpu.CompilerParams(
            dimension_semantics=("parallel", "parallel"),
        ),
    )(tbl, logits, top, bot)


# speedup vs baseline: 1608.6267x; 1.3290x over previous
import jax
import jax.numpy as jnp
import numpy as np
from jax.experimental import pallas as pl
from jax.experimental.pallas import tpu as pltpu

_NUM_BINS = 15
_NUM_CLASSES = 19
_NBR = 9
_H = 512
_W = 512
_HT = 64


def _body(tbl_ref, logits_ref, top_ref, bot_ref, out_ref):
    h = pl.program_id(1)
    n_h = pl.num_programs(1)
    ht = _HT

    m = logits_ref[0, 0]
    mt = top_ref[0, 0, 0]
    mb = bot_ref[0, 0, 0]
    for c in range(1, _NUM_CLASSES):
        m = jnp.maximum(m, logits_ref[0, c])
        mt = jnp.maximum(mt, top_ref[0, c, 0])
        mb = jnp.maximum(mb, bot_ref[0, c, 0])

    yh = jnp.zeros((ht, _W), jnp.int32)
    yt = jnp.zeros((1, _W), jnp.int32)
    yb = jnp.zeros((1, _W), jnp.int32)
    s = jnp.zeros((ht, _W), jnp.float32)
    for c in range(_NUM_CLASSES - 1, -1, -1):
        x = logits_ref[0, c]
        yh = jnp.where(x == m, c, yh)
        e = jnp.exp(x - m)
        out_ref[0, c] = e
        s = s + e
        yt = jnp.where(top_ref[0, c, 0] == mt, c, yt)
        yb = jnp.where(bot_ref[0, c, 0] == mb, c, yb)
    inv_s = 1.0 / s

    top_valid = h > 0
    bot_valid = h < n_h - 1
    yt = jnp.where(top_valid, yt, -1)
    yb = jnp.where(bot_valid, yb, -1)

    yext = jnp.concatenate([yt, yh, yb], axis=0)
    zero_col = jnp.zeros((ht + 2, 1), jnp.int32)
    region = yext >> 3
    field = (yext & 7) << 2
    onebit = jnp.left_shift(jnp.ones_like(yext), field)
    pos_words = []
    for r in range(3):
        br = jnp.where(region == r, onebit, 0)
        brl = jnp.concatenate([br[:, 1:], zero_col], axis=1)
        brr = jnp.concatenate([zero_col, br[:, :-1]], axis=1)
        rowsum = br + brl + brr
        pw = (rowsum[0:ht] + rowsum[1:ht + 1] + rowsum[2:ht + 2]
              - br[1:ht + 1])
        pos_words.append(pw)

    grow = h * ht + jax.lax.broadcasted_iota(jnp.int32, (ht, _W), 0)
    gcol = jax.lax.broadcasted_iota(jnp.int32, (ht, _W), 1)
    vert = (1 + (grow > 0).astype(jnp.int32)
            + (grow < _H - 1).astype(jnp.int32))
    horiz = (1 + (gcol > 0).astype(jnp.int32)
             + (gcol < _W - 1).astype(jnp.int32))
    ones_cnt = vert * horiz - 1

    csum = jnp.zeros((ht, _W), jnp.float32)
    for c in range(_NUM_CLASSES):
        p = out_ref[0, c] * inv_s
        bin_i = jnp.minimum((p * float(_NUM_BINS)).astype(jnp.int32),
                            _NUM_BINS - 1)
        pos = (pos_words[c >> 3] >> ((c & 7) << 2)) & 15
        match = jnp.where(yh == c, pos, ones_cnt - pos)
        comb = match * _NUM_BINS + bin_i

        row = tbl_ref[c]
        src_a = jnp.broadcast_to(row[0:1, :], (ht, 128))
        src_b = jnp.broadcast_to(row[1:2, :], (ht, 128))
        pieces = []
        for j in range(_W // 128):
            idx = comb[:, j * 128:(j + 1) * 128]
            ga = jnp.take_along_axis(src_a, jnp.minimum(idx, 127), axis=1,
                                     mode="promise_in_bounds")
            gb = jnp.take_along_axis(src_b, jnp.maximum(idx - 128, 0),
                                     axis=1, mode="promise_in_bounds")
            pieces.append(jnp.where(idx < 128, ga, gb))
        val = jnp.concatenate(pieces, axis=1)
        out_ref[0, c] = val
        csum = csum + val

    inv = 1.0 / jnp.where(csum == 0.0, 1.0, csum)
    for c in range(_NUM_CLASSES):
        out_ref[0, c] = out_ref[0, c] * inv


def kernel(logits, val_freqs):
    b, nc, hh, ww = logits.shape
    n_h = hh // _HT

    flat = val_freqs.reshape(_NUM_CLASSES, _NBR * _NUM_BINS).astype(jnp.float32)
    tbl = jnp.zeros((_NUM_CLASSES, 2, 128), jnp.float32)
    tbl = tbl.at[:, 0, :].set(flat[:, :128])
    tbl = tbl.at[:, 1, :flat.shape[1] - 128].set(flat[:, 128:])

    tidx = np.arange(n_h) * _HT - 1
    tidx[0] = 0
    bidx = np.arange(n_h) * _HT + _HT
    bidx[-1] = hh - 1
    top = logits[:, :, tidx, :].reshape(b, nc, n_h, 1, ww)
    bot = logits[:, :, bidx, :].reshape(b, nc, n_h, 1, ww)

    return pl.pallas_call(
        _body,
        grid=(b, n_h),
        in_specs=[
            pl.BlockSpec((_NUM_CLASSES, 2, 128), lambda b_, h_: (0, 0, 0)),
            pl.BlockSpec((1, _NUM_CLASSES, _HT, _W),
                         lambda b_, h_: (b_, 0, h_, 0)),
            pl.BlockSpec((1, _NUM_CLASSES, 1, 1, _W),
                         lambda b_, h_: (b_, 0, h_, 0, 0)),
            pl.BlockSpec((1, _NUM_CLASSES, 1, 1, _W),
                         lambda b_, h_: (b_, 0, h_, 0, 0)),
        ],
        out_specs=pl.BlockSpec((1, _NUM_CLASSES, _HT, _W),
                               lambda b_, h_: (b_, 0, h_, 0)),
        out_shape=jax.ShapeDtypeStruct((b, nc, hh, ww), jnp.float32),
        compiler_params=pltpu.CompilerParams(
            dimension_semantics=("parallel", "parallel"),
        ),
    )(tbl, logits, top, bot)


# speedup vs baseline: 1633.7696x; 1.0156x over previous
import jax
import jax.numpy as jnp
import numpy as np
from jax.experimental import pallas as pl
from jax.experimental.pallas import tpu as pltpu

_NUM_BINS = 15
_NUM_CLASSES = 19
_NBR = 9
_H = 512
_W = 512
_HT = 128


def _body(tbl_ref, logits_ref, top_ref, bot_ref, out_ref):
    h = pl.program_id(1)
    n_h = pl.num_programs(1)
    ht = _HT

    m = logits_ref[0, 0]
    mt = top_ref[0, 0, 0]
    mb = bot_ref[0, 0, 0]
    for c in range(1, _NUM_CLASSES):
        m = jnp.maximum(m, logits_ref[0, c])
        mt = jnp.maximum(mt, top_ref[0, c, 0])
        mb = jnp.maximum(mb, bot_ref[0, c, 0])

    yh = jnp.zeros((ht, _W), jnp.int32)
    yt = jnp.zeros((1, _W), jnp.int32)
    yb = jnp.zeros((1, _W), jnp.int32)
    s = jnp.zeros((ht, _W), jnp.float32)
    for c in range(_NUM_CLASSES - 1, -1, -1):
        x = logits_ref[0, c]
        yh = jnp.where(x == m, c, yh)
        e = jnp.exp(x - m)
        out_ref[0, c] = e
        s = s + e
        yt = jnp.where(top_ref[0, c, 0] == mt, c, yt)
        yb = jnp.where(bot_ref[0, c, 0] == mb, c, yb)
    inv_s = 1.0 / s

    top_valid = h > 0
    bot_valid = h < n_h - 1
    yt = jnp.where(top_valid, yt, -1)
    yb = jnp.where(bot_valid, yb, -1)

    yext = jnp.concatenate([yt, yh, yb], axis=0)
    zero_col = jnp.zeros((ht + 2, 1), jnp.int32)
    region = yext >> 3
    field = (yext & 7) << 2
    onebit = jnp.left_shift(jnp.ones_like(yext), field)
    pos_words = []
    for r in range(3):
        br = jnp.where(region == r, onebit, 0)
        brl = jnp.concatenate([br[:, 1:], zero_col], axis=1)
        brr = jnp.concatenate([zero_col, br[:, :-1]], axis=1)
        rowsum = br + brl + brr
        pw = (rowsum[0:ht] + rowsum[1:ht + 1] + rowsum[2:ht + 2]
              - br[1:ht + 1])
        pos_words.append(pw)

    grow = h * ht + jax.lax.broadcasted_iota(jnp.int32, (ht, _W), 0)
    gcol = jax.lax.broadcasted_iota(jnp.int32, (ht, _W), 1)
    vert = (1 + (grow > 0).astype(jnp.int32)
            + (grow < _H - 1).astype(jnp.int32))
    horiz = (1 + (gcol > 0).astype(jnp.int32)
             + (gcol < _W - 1).astype(jnp.int32))
    ones_cnt = vert * horiz - 1

    csum = jnp.zeros((ht, _W), jnp.float32)
    for c in range(_NUM_CLASSES):
        p = out_ref[0, c] * inv_s
        bin_i = jnp.minimum((p * float(_NUM_BINS)).astype(jnp.int32),
                            _NUM_BINS - 1)
        pos = (pos_words[c >> 3] >> ((c & 7) << 2)) & 15
        match = jnp.where(yh == c, pos, ones_cnt - pos)
        comb = match * _NUM_BINS + bin_i

        row = tbl_ref[c]
        src_a = jnp.broadcast_to(row[0:1, :], (ht, 128))
        src_b = jnp.broadcast_to(row[1:2, :], (ht, 128))
        pieces = []
        for j in range(_W // 128):
            idx = comb[:, j * 128:(j + 1) * 128]
            ga = jnp.take_along_axis(src_a, jnp.minimum(idx, 127), axis=1,
                                     mode="promise_in_bounds")
            gb = jnp.take_along_axis(src_b, jnp.maximum(idx - 128, 0),
                                     axis=1, mode="promise_in_bounds")
            pieces.append(jnp.where(idx < 128, ga, gb))
        val = jnp.concatenate(pieces, axis=1)
        out_ref[0, c] = val
        csum = csum + val

    inv = 1.0 / jnp.where(csum == 0.0, 1.0, csum)
    for c in range(_NUM_CLASSES):
        out_ref[0, c] = out_ref[0, c] * inv


def kernel(logits, val_freqs):
    b, nc, hh, ww = logits.shape
    n_h = hh // _HT

    flat = val_freqs.reshape(_NUM_CLASSES, _NBR * _NUM_BINS).astype(jnp.float32)
    tbl = jnp.zeros((_NUM_CLASSES, 2, 128), jnp.float32)
    tbl = tbl.at[:, 0, :].set(flat[:, :128])
    tbl = tbl.at[:, 1, :flat.shape[1] - 128].set(flat[:, 128:])

    tidx = np.arange(n_h) * _HT - 1
    tidx[0] = 0
    bidx = np.arange(n_h) * _HT + _HT
    bidx[-1] = hh - 1
    top = logits[:, :, tidx, :].reshape(b, nc, n_h, 1, ww)
    bot = logits[:, :, bidx, :].reshape(b, nc, n_h, 1, ww)

    return pl.pallas_call(
        _body,
        grid=(b, n_h),
        in_specs=[
            pl.BlockSpec((_NUM_CLASSES, 2, 128), lambda b_, h_: (0, 0, 0)),
            pl.BlockSpec((1, _NUM_CLASSES, _HT, _W),
                         lambda b_, h_: (b_, 0, h_, 0)),
            pl.BlockSpec((1, _NUM_CLASSES, 1, 1, _W),
                         lambda b_, h_: (b_, 0, h_, 0, 0)),
            pl.BlockSpec((1, _NUM_CLASSES, 1, 1, _W),
                         lambda b_, h_: (b_, 0, h_, 0, 0)),
        ],
        out_specs=pl.BlockSpec((1, _NUM_CLASSES, _HT, _W),
                               lambda b_, h_: (b_, 0, h_, 0)),
        out_shape=jax.ShapeDtypeStruct((b, nc, hh, ww), jnp.float32),
        compiler_params=pltpu.CompilerParams(
            dimension_semantics=("parallel", "parallel"),
        ),
    )(tbl, logits, top, bot)
